# baseline (speedup 1.0000x reference)
import functools

import jax
import jax.numpy as jnp
from jax import lax
from jax.experimental import pallas as pl
from jax.experimental.pallas import tpu as pltpu
from jax.experimental.pallas import tpu_sc as plsc

B = 8
P = 8
N = 8192
G = 64
V = G ** 3
CHUNKS = N // 16
PAIRS_PER_WORKER = (B * P) // 32


def _sc_body(pts_hbm, planes_hbm, cp_hbm, vox_hbm, out_hbm,
             px_v, py_v, pz_v, tx_v, ty_v, tz_v,
             ia_v, ib_v, ic_v, iv_v, ga_v, gb_v, gc_v, gv_v,
             plane_v, acc_v, sem_a, sem_b, sem_c, sem_v):
    wid = lax.axis_index("s") * 2 + lax.axis_index("c")
    b = wid // 4

    pbase = b * (3 * N)
    pltpu.sync_copy(pts_hbm.at[pl.ds(pl.multiple_of(pbase, N), N)], px_v)
    pltpu.sync_copy(pts_hbm.at[pl.ds(pl.multiple_of(pbase + N, N), N)], py_v)
    pltpu.sync_copy(pts_hbm.at[pl.ds(pl.multiple_of(pbase + 2 * N, N), N)], pz_v)

    acc = jnp.zeros((16,), jnp.float32)
    base_off = b * V

    for k in range(PAIRS_PER_WORKER):
        pair = wid * PAIRS_PER_WORKER + k
        pltpu.sync_copy(
            planes_hbm.at[pl.ds(pl.multiple_of(pair * 64, 64), 64)], plane_v)
        nx = plane_v[pl.ds(0, 16)]
        ny = plane_v[pl.ds(16, 16)]
        nz = plane_v[pl.ds(32, 16)]
        dd = plane_v[pl.ds(48, 16)]
        inv2 = 2.0 / (nx * nx + ny * ny + nz * nz)

        def body_a(r, carry):
            sl = pl.ds(pl.multiple_of(r * 16, 16), 16)
            px = px_v[sl]
            py = py_v[sl]
            pz = pz_v[sl]
            f = (px * nx + py * ny + pz * nz + dd) * inv2
            tx = px - f * nx
            ty = py - f * ny
            tz = pz - f * nz
            tx_v[sl] = tx
            ty_v[sl] = ty
            tz_v[sl] = tz

            def ceil_i(t):
                z = (t + 0.5) * float(G) - 0.5
                i = z.astype(jnp.int32)
                return jnp.where(z > i.astype(jnp.float32), i + 1, i)

            flat = ceil_i(tx) * (G * G) + ceil_i(ty) * G + ceil_i(tz)
            flat = jnp.minimum(jnp.maximum(flat, 0), V - 1)
            base = flat + base_off
            b3 = base * 3
            ia_v[sl] = b3
            ib_v[sl] = b3 + 1
            ic_v[sl] = b3 + 2
            iv_v[sl] = base
            return carry

        lax.fori_loop(0, CHUNKS, body_a, 0, unroll=8)

        da = pltpu.async_copy(cp_hbm.at[ia_v], ga_v, sem_a)
        db = pltpu.async_copy(cp_hbm.at[ib_v], gb_v, sem_b)
        dc = pltpu.async_copy(cp_hbm.at[ic_v], gc_v, sem_c)
        dv = pltpu.async_copy(vox_hbm.at[iv_v], gv_v, sem_v)
        da.wait()
        db.wait()
        dc.wait()
        dv.wait()

        def body_c(r, a):
            sl = pl.ds(pl.multiple_of(r * 16, 16), 16)
            dx = tx_v[sl] - ga_v[sl]
            dy = ty_v[sl] - gb_v[sl]
            dz = tz_v[sl] - gc_v[sl]
            m = 1.0 - gv_v[sl]
            return a + (m * m) * (dx * dx + dy * dy + dz * dz)

        acc = lax.fori_loop(0, CHUNKS, body_c, acc, unroll=8)

    acc_v[...] = acc
    pltpu.sync_copy(acc_v, out_hbm.at[pl.ds(pl.multiple_of(wid * 16, 16), 16)])


@jax.jit
def _sc_loss(pts_t, planes_pad, cp_flat, vox_flat):
    mesh = plsc.VectorSubcoreMesh(core_axis_name="c", subcore_axis_name="s")
    f32 = jnp.float32
    i32 = jnp.int32
    kern = functools.partial(
        pl.kernel,
        mesh=mesh,
        out_type=jax.ShapeDtypeStruct((32 * 16,), f32),
        scratch_types=[
            pltpu.VMEM((N,), f32),
            pltpu.VMEM((N,), f32),
            pltpu.VMEM((N,), f32),
            pltpu.VMEM((N,), f32),
            pltpu.VMEM((N,), f32),
            pltpu.VMEM((N,), f32),
            pltpu.VMEM((N,), i32),
            pltpu.VMEM((N,), i32),
            pltpu.VMEM((N,), i32),
            pltpu.VMEM((N,), i32),
            pltpu.VMEM((N,), f32),
            pltpu.VMEM((N,), f32),
            pltpu.VMEM((N,), f32),
            pltpu.VMEM((N,), f32),
            pltpu.VMEM((64,), f32),
            pltpu.VMEM((16,), f32),
            pltpu.SemaphoreType.DMA,
            pltpu.SemaphoreType.DMA,
            pltpu.SemaphoreType.DMA,
            pltpu.SemaphoreType.DMA,
        ],
    )(_sc_body)
    return kern(pts_t, planes_pad, cp_flat, vox_flat)


def kernel(voxel, points, closest_points, planes):
    pts_t = jnp.transpose(points, (0, 2, 1)).reshape(-1)
    planes_pad = jnp.broadcast_to(
        planes.reshape(B * P, 4)[:, :, None], (B * P, 4, 16)).reshape(-1)
    cp_flat = closest_points.reshape(-1)
    vox_flat = voxel.reshape(-1)
    partial = _sc_loss(pts_t, planes_pad, cp_flat, vox_flat)
    return jnp.sum(partial) / (B * P)


# speedup vs baseline: 24.2634x; 24.2634x over previous
import functools

import jax
import jax.numpy as jnp
from jax import lax
from jax.experimental import pallas as pl
from jax.experimental.pallas import tpu as pltpu
from jax.experimental.pallas import tpu_sc as plsc

B = 8
P = 8
N = 8192
G = 64
V = G ** 3
CHUNKS = N // 16
PAIRS_PER_WORKER = (B * P) // 32


def _sc_body(px_hbm, py_hbm, pz_hbm, planes_hbm, cpx_hbm, cpy_hbm, cpz_hbm,
             m2_hbm, out_hbm,
             px_v, py_v, pz_v, tx_v, ty_v, tz_v,
             iv_v, ga_v, gb_v, gc_v, gv_v,
             plane_v, acc_v, sem_a, sem_b, sem_c, sem_v):
    wid = lax.axis_index("s") * 2 + lax.axis_index("c")
    b = wid // 4

    pbase = b * N
    pltpu.sync_copy(px_hbm.at[pl.ds(pl.multiple_of(pbase, N), N)], px_v)
    pltpu.sync_copy(py_hbm.at[pl.ds(pl.multiple_of(pbase, N), N)], py_v)
    pltpu.sync_copy(pz_hbm.at[pl.ds(pl.multiple_of(pbase, N), N)], pz_v)

    acc = jnp.zeros((16,), jnp.float32)
    base_off = b * V

    for k in range(PAIRS_PER_WORKER):
        pair = wid * PAIRS_PER_WORKER + k
        pltpu.sync_copy(
            planes_hbm.at[pl.ds(pl.multiple_of(pair * 64, 64), 64)], plane_v)
        nx = plane_v[pl.ds(0, 16)]
        ny = plane_v[pl.ds(16, 16)]
        nz = plane_v[pl.ds(32, 16)]
        dd = plane_v[pl.ds(48, 16)]
        inv2 = 2.0 / (nx * nx + ny * ny + nz * nz)

        def body_a(r, carry):
            sl = pl.ds(pl.multiple_of(r * 16, 16), 16)
            px = px_v[sl]
            py = py_v[sl]
            pz = pz_v[sl]
            f = (px * nx + py * ny + pz * nz + dd) * inv2
            tx = px - f * nx
            ty = py - f * ny
            tz = pz - f * nz
            tx_v[sl] = tx
            ty_v[sl] = ty
            tz_v[sl] = tz

            def ceil_i(t):
                z = (t + 0.5) * float(G) - 0.5
                i = z.astype(jnp.int32)
                return jnp.where(z > i.astype(jnp.float32), i + 1, i)

            flat = ceil_i(tx) * (G * G) + ceil_i(ty) * G + ceil_i(tz)
            flat = jnp.minimum(jnp.maximum(flat, 0), V - 1)
            iv_v[sl] = flat + base_off
            return carry

        lax.fori_loop(0, CHUNKS, body_a, 0, unroll=8)

        da = pltpu.async_copy(cpx_hbm.at[iv_v], ga_v, sem_a)
        db = pltpu.async_copy(cpy_hbm.at[iv_v], gb_v, sem_b)
        dc = pltpu.async_copy(cpz_hbm.at[iv_v], gc_v, sem_c)
        dv = pltpu.async_copy(m2_hbm.at[iv_v], gv_v, sem_v)
        da.wait()
        db.wait()
        dc.wait()
        dv.wait()

        def body_c(r, a):
            sl = pl.ds(pl.multiple_of(r * 16, 16), 16)
            dx = tx_v[sl] - ga_v[sl]
            dy = ty_v[sl] - gb_v[sl]
            dz = tz_v[sl] - gc_v[sl]
            return a + gv_v[sl] * (dx * dx + dy * dy + dz * dz)

        acc = lax.fori_loop(0, CHUNKS, body_c, acc, unroll=8)

    acc_v[...] = acc
    pltpu.sync_copy(acc_v, out_hbm.at[pl.ds(pl.multiple_of(wid * 16, 16), 16)])


@jax.jit
def _sc_loss(px, py, pz, planes_pad, cpx, cpy, cpz, m2):
    mesh = plsc.VectorSubcoreMesh(core_axis_name="c", subcore_axis_name="s")
    f32 = jnp.float32
    i32 = jnp.int32
    kern = functools.partial(
        pl.kernel,
        mesh=mesh,
        out_type=jax.ShapeDtypeStruct((32 * 16,), f32),
        scratch_types=[
            pltpu.VMEM((N,), f32),
            pltpu.VMEM((N,), f32),
            pltpu.VMEM((N,), f32),
            pltpu.VMEM((N,), f32),
            pltpu.VMEM((N,), f32),
            pltpu.VMEM((N,), f32),
            pltpu.VMEM((N,), i32),
            pltpu.VMEM((N,), f32),
            pltpu.VMEM((N,), f32),
            pltpu.VMEM((N,), f32),
            pltpu.VMEM((N,), f32),
            pltpu.VMEM((64,), f32),
            pltpu.VMEM((16,), f32),
            pltpu.SemaphoreType.DMA,
            pltpu.SemaphoreType.DMA,
            pltpu.SemaphoreType.DMA,
            pltpu.SemaphoreType.DMA,
        ],
    )(_sc_body)
    return kern(px, py, pz, planes_pad, cpx, cpy, cpz, m2)


def kernel(voxel, points, closest_points, planes):
    s = 1.0 + 0.0 * jnp.sum(planes)
    px = (points[:, :, 0] * s).reshape(-1)
    py = (points[:, :, 1] * s).reshape(-1)
    pz = (points[:, :, 2] * s).reshape(-1)
    cpx = (closest_points[:, :, 0] * s).reshape(-1)
    cpy = (closest_points[:, :, 1] * s).reshape(-1)
    cpz = (closest_points[:, :, 2] * s).reshape(-1)
    mask = 1.0 - voxel
    m2 = (mask * mask).reshape(-1)
    planes_pad = (jnp.broadcast_to(
        planes.reshape(B * P, 4)[:, :, None], (B * P, 4, 16)) * s).reshape(-1)
    partial = _sc_loss(px, py, pz, planes_pad, cpx, cpy, cpz, m2)
    return jnp.sum(partial) / (B * P)
